# baseline (speedup 1.0000x reference)
import jax
import jax.numpy as jnp
from jax.experimental import pallas as pl


def _row_stats(x):
    m = x.mean(axis=-1, keepdims=True)
    s = x.std(axis=-1, keepdims=True, ddof=1)
    return m, s


def _fuse_body(x1_ref, x2_ref, x3_ref, st_ref, out_ref):
    del st_ref
    x1 = jnp.abs(x1_ref[...])
    x2 = x2_ref[...]
    x3 = x3_ref[...]
    m1, s1 = _row_stats(x1)
    m2, s2 = _row_stats(x2)
    m3, s3 = _row_stats(x3)
    z1 = (x1 - m1) / (s1 + 1e-6)
    z2 = (x2 - m2) / (s2 + 1e-6)
    z3 = (x3 - m3) / (s3 + 1e-6)
    out_ref[...] = (z1 + z2 + z3) / 3.0


def kernel(shap, attn, symb):
    B, N = shap.shape
    m1, s1 = _row_stats(jnp.abs(shap))
    m2, s2 = _row_stats(attn)
    m3, s3 = _row_stats(symb)
    z = jnp.zeros_like(m1)
    st = jnp.concatenate([m1, s1, m2, s2, m3, s3, z, z], axis=1)

    R = 8
    fused = pl.pallas_call(
        _fuse_body,
        grid=(B // R,),
        in_specs=[
            pl.BlockSpec((R, N), lambda i: (i, 0)),
            pl.BlockSpec((R, N), lambda i: (i, 0)),
            pl.BlockSpec((R, N), lambda i: (i, 0)),
            pl.BlockSpec((R, 8), lambda i: (i, 0)),
        ],
        out_specs=pl.BlockSpec((R, N), lambda i: (i, 0)),
        out_shape=jax.ShapeDtypeStruct((B, N), jnp.float32),
    )(shap, attn, symb, st)

    k = max(1, N // 20)
    _, top_bins = jax.lax.top_k(fused, k)
    return fused, top_bins


# speedup vs baseline: 3.7504x; 3.7504x over previous
import functools

import jax
import jax.numpy as jnp
import numpy as np
from jax import lax
from jax.experimental import pallas as pl
from jax.experimental.pallas import tpu as pltpu
from jax.experimental.pallas import tpu_sc as plsc

B, N = 128, 32768
K = N // 20
KOUT = 1664
CAPE = 1792
NVC = CAPE // 16
CBUF = CAPE + 16
NROWVREGS = N // 16
ROWS_PER_W = B // 32

_MSB = np.uint32(0x80000000)
_ALL1 = np.uint32(0xFFFFFFFF)


def _row_stats(x):
    m = x.mean(axis=-1, keepdims=True)
    s = x.std(axis=-1, keepdims=True, ddof=1)
    return m, s


def _fuse_body(x1_ref, x2_ref, x3_ref, st_ref, out_ref, key_ref):
    m1 = st_ref[:, 0:1]
    s1 = st_ref[:, 1:2]
    m2 = st_ref[:, 2:3]
    s2 = st_ref[:, 3:4]
    m3 = st_ref[:, 4:5]
    s3 = st_ref[:, 5:6]
    z1 = (jnp.abs(x1_ref[...]) - m1) / (s1 + 1e-6)
    z2 = (x2_ref[...] - m2) / (s2 + 1e-6)
    z3 = (x3_ref[...] - m3) / (s3 + 1e-6)
    f = (z1 + z2 + z3) / 3.0
    out_ref[...] = f
    u = lax.bitcast_convert_type(f, jnp.uint32)
    key_ref[...] = jnp.where(f < 0.0, u ^ _ALL1, u | _MSB)


def _iota16():
    return lax.broadcasted_iota(jnp.int32, (16,), 0)


def _splat(scalar):
    return jnp.broadcast_to(scalar, (16,))


def _find_threshold(hist, ngroups, krem, z16):
    zero = np.int32(0)

    def body(j, carry):
        run, dacc, aacc, cacc = carry
        jj = ngroups - 1 - j
        t = hist[pl.ds(16 * jj, 16)]
        for l in range(1, 16):
            t = t + hist[pl.ds(l * 2048 + 16 * jj, 16)]
        rt = lax.rev(t, (0,))
        cum = plsc.cumsum(rt)
        incl = _splat(run) + cum
        prev = incl - rt
        kv = _splat(krem)
        hit = (incl >= kv) & (prev < kv)
        digits_desc = _splat(16 * jj + 15) - _iota16()
        dacc = jnp.maximum(dacc, jnp.where(hit, digits_desc + 1, zero))
        aacc = jnp.maximum(aacc, jnp.where(hit, prev + 1, zero))
        cacc = jnp.maximum(cacc, jnp.where(hit, rt + 1, zero))
        return incl[15], dacc, aacc, cacc

    _, dacc, aacc, cacc = lax.fori_loop(0, ngroups, body,
                                        (zero, z16, z16, z16))
    return jnp.max(dacc) - 1, jnp.max(aacc) - 1, jnp.max(cacc) - 1


def _clear_hist(hist, nbins, z16):
    def body(j, _):
        for l in range(16):
            hist[pl.ds(l * 2048 + 16 * j, 16)] = z16
        return 0

    lax.fori_loop(0, nbins // 16, body, 0, unroll=4)


def _radix_sort(ck, ci, ck2, ci2, cnt, offs, z16):
    bufs = [(ck, ci, ck2, ci2), (ck2, ci2, ck, ci)]
    for p in range(4):
        srck, srci, dstk, dsti = bufs[p % 2]
        sh = np.int32(8 * p)

        def pclr(g, _):
            cnt[pl.ds(16 * g, 16)] = z16
            return 0

        lax.fori_loop(0, 16, pclr, 0, unroll=4)

        def pha(v, _):
            key = srck[pl.ds(16 * v, 16)]
            d = np.int32(255) - (lax.shift_right_logical(key, sh)
                                 & np.int32(255))
            c, last = plsc.scan_count(d)
            plsc.addupdate_scatter(cnt, [d], c, mask=last)
            return 0

        lax.fori_loop(0, NVC, pha, 0, unroll=2)

        def phb(g, run):
            t = cnt[pl.ds(16 * g, 16)]
            c = plsc.cumsum(t)
            offs[pl.ds(16 * g, 16)] = _splat(run) + c - t
            return run + c[15]

        lax.fori_loop(0, 16, phb, np.int32(0))

        def phc(v, _):
            key = srck[pl.ds(16 * v, 16)]
            idx = srci[pl.ds(16 * v, 16)]
            d = np.int32(255) - (lax.shift_right_logical(key, sh)
                                 & np.int32(255))
            c, last = plsc.scan_count(d)
            base = plsc.load_gather(offs, [d])
            pos = base + c - 1
            plsc.store_scatter(dstk, [pos], key)
            plsc.store_scatter(dsti, [pos], idx)
            plsc.addupdate_scatter(offs, [d], c, mask=last)
            return 0

        lax.fori_loop(0, NVC, phc, 0)


def _topk_body(keys_hbm, out_hbm, buf_k, hist, ck, ci, ck2, ci2, cei, offs,
               cnt, aux):
    wid = lax.axis_index("s") * 2 + lax.axis_index("c")
    lanes = _iota16()
    z16 = jnp.zeros_like(lanes)
    ones = z16 + 1

    def row_task(t, _):
        r = wid * ROWS_PER_W + t
        pltpu.sync_copy(keys_hbm.at[r], buf_k)

        _clear_hist(hist, 1024, z16)

        def p1(i, _):
            key = buf_k[pl.ds(16 * i, 16)]
            dig = lax.shift_right_logical(key, np.uint32(22)).astype(jnp.int32)
            plsc.addupdate_scatter(hist, [lanes * 2048 + dig], ones)
            return 0

        lax.fori_loop(0, NROWVREGS, p1, 0, unroll=4)
        d1, a1, c1 = _find_threshold(hist, 64, np.int32(K), z16)
        ncand = a1 + c1
        direct = ncand <= np.int32(CAPE)

        def pclr(v, _):
            ck[pl.ds(16 * v, 16)] = z16
            ci[pl.ds(16 * v, 16)] = z16
            return 0

        lax.fori_loop(0, NVC + 1, pclr, 0, unroll=4)

        @pl.when(direct)
        def _():
            lim = lax.shift_left(d1, np.int32(22)).astype(jnp.uint32)
            lv = _splat(lim)

            def pcol(i, nc):
                key = buf_k[pl.ds(16 * i, 16)]
                ge = key >= lv
                plsc.store_compressed(ck.at[pl.ds(nc, 16)],
                                      plsc.bitcast(key, jnp.int32), mask=ge)
                idxv = _splat(16 * i) + lanes
                plsc.store_compressed(ci.at[pl.ds(nc, 16)], idxv, mask=ge)
                return nc + plsc.all_reduce_population_count(ge)[0]

            lax.fori_loop(0, NROWVREGS, pcol, np.int32(0), unroll=2)
            aux[pl.ds(0, 16)] = z16

        @pl.when(jnp.logical_not(direct))
        def _():
            rem1 = np.int32(K) - a1
            _clear_hist(hist, 2048, z16)
            d1u = d1.astype(jnp.uint32)

            def p2(i, _):
                key = buf_k[pl.ds(16 * i, 16)]
                m = lax.shift_right_logical(key, np.uint32(22)) == d1u
                dig = (lax.shift_right_logical(key, np.uint32(11))
                       & np.uint32(0x7FF)).astype(jnp.int32)
                plsc.addupdate_scatter(hist, [lanes * 2048 + dig], ones,
                                       mask=m)
                return 0

            lax.fori_loop(0, NROWVREGS, p2, 0, unroll=2)
            d2, a2, _c2 = _find_threshold(hist, 128, rem1, z16)
            rem2 = rem1 - a2

            _clear_hist(hist, 2048, z16)
            pref = ((lax.shift_left(d1, np.int32(11)) | d2)
                    .astype(jnp.uint32))

            def p3(i, _):
                key = buf_k[pl.ds(16 * i, 16)]
                m = lax.shift_right_logical(key, np.uint32(11)) == pref
                dig = (key & np.uint32(0x7FF)).astype(jnp.int32)
                plsc.addupdate_scatter(hist, [lanes * 2048 + dig], ones,
                                       mask=m)
                return 0

            lax.fori_loop(0, NROWVREGS, p3, 0, unroll=2)
            d3, a3, _c3 = _find_threshold(hist, 128, rem2, z16)
            need = rem2 - a3
            tkey = ((lax.shift_left(d1, np.int32(22))
                     | lax.shift_left(d2, np.int32(11)) | d3)
                    .astype(jnp.uint32))
            tk = _splat(tkey)

            def pcomp(i, carry):
                ngt, neq = carry
                key = buf_k[pl.ds(16 * i, 16)]
                idxv = _splat(16 * i) + lanes
                gt = key > tk
                eq = key == tk
                plsc.store_compressed(ck.at[pl.ds(ngt, 16)],
                                      plsc.bitcast(key, jnp.int32), mask=gt)
                plsc.store_compressed(ci.at[pl.ds(ngt, 16)], idxv, mask=gt)
                neqc = jnp.minimum(neq, np.int32(CAPE - 16))
                plsc.store_compressed(cei.at[pl.ds(neqc, 16)], idxv, mask=eq)
                ngt = ngt + plsc.all_reduce_population_count(gt)[0]
                neq = neq + plsc.all_reduce_population_count(eq)[0]
                return ngt, neq

            lax.fori_loop(0, NROWVREGS, pcomp, (np.int32(0), np.int32(0)))
            aux[pl.ds(0, 16)] = _splat(need)

        _radix_sort(ck, ci, ck2, ci2, cnt, offs, z16)

        need = aux[pl.ds(0, 16)][0]
        ngt = np.int32(K) - need

        def papp(v, _):
            @pl.when(16 * v < need)
            def _():
                ci[pl.ds(ngt + 16 * v, 16)] = cei[pl.ds(16 * v, 16)]
            return 0

        lax.fori_loop(0, NVC, papp, 0)
        pltpu.sync_copy(ci.at[pl.ds(0, KOUT)], out_hbm.at[r])
        return 0

    lax.fori_loop(0, ROWS_PER_W, row_task, 0)


_sc_mesh = plsc.VectorSubcoreMesh(core_axis_name="c", subcore_axis_name="s")

_topk_call = functools.partial(
    pl.kernel,
    out_type=jax.ShapeDtypeStruct((B, KOUT), jnp.int32),
    mesh=_sc_mesh,
    scratch_types=[
        pltpu.VMEM((N,), jnp.uint32),
        pltpu.VMEM((16 * 2048,), jnp.int32),
        pltpu.VMEM((CBUF,), jnp.int32),
        pltpu.VMEM((CBUF,), jnp.int32),
        pltpu.VMEM((CBUF,), jnp.int32),
        pltpu.VMEM((CBUF,), jnp.int32),
        pltpu.VMEM((CBUF,), jnp.int32),
        pltpu.VMEM((256,), jnp.int32),
        pltpu.VMEM((256,), jnp.int32),
        pltpu.VMEM((16,), jnp.int32),
    ],
    compiler_params=pltpu.CompilerParams(needs_layout_passes=False),
)(_topk_body)


def kernel(shap, attn, symb):
    m1, s1 = _row_stats(jnp.abs(shap))
    m2, s2 = _row_stats(attn)
    m3, s3 = _row_stats(symb)
    z = jnp.zeros_like(m1)
    st = jnp.concatenate([m1, s1, m2, s2, m3, s3, z, z], axis=1)

    R = 8
    fused, keys = pl.pallas_call(
        _fuse_body,
        grid=(B // R,),
        in_specs=[
            pl.BlockSpec((R, N), lambda i: (i, 0)),
            pl.BlockSpec((R, N), lambda i: (i, 0)),
            pl.BlockSpec((R, N), lambda i: (i, 0)),
            pl.BlockSpec((R, 8), lambda i: (i, 0)),
        ],
        out_specs=[
            pl.BlockSpec((R, N), lambda i: (i, 0)),
            pl.BlockSpec((R, N), lambda i: (i, 0)),
        ],
        out_shape=[
            jax.ShapeDtypeStruct((B, N), jnp.float32),
            jax.ShapeDtypeStruct((B, N), jnp.uint32),
        ],
    )(shap, attn, symb, st)

    top_bins = _topk_call(keys)[:, :K]
    return fused, top_bins
